# baseline (speedup 1.0000x reference)
import functools

import jax
import jax.numpy as jnp
from jax import lax
from jax.experimental import pallas as pl
from jax.experimental.pallas import tpu as pltpu
from jax.experimental.pallas import tpu_sc as plsc

NROW = 4096
SEQ = 200
D = 64
DP = 128
NC = 2
NS = 16
NW = NC * NS
R_PER_W = NROW // NW
NBUF = 4

_mesh = plsc.VectorSubcoreMesh(core_axis_name="c", subcore_axis_name="s")


@functools.partial(
    pl.kernel,
    mesh=_mesh,
    out_type=jax.ShapeDtypeStruct((NROW, SEQ, DP), jnp.float32),
    scratch_types=[
        [pltpu.VMEM((SEQ,), jnp.int32) for _ in range(NBUF)],
        [pltpu.VMEM((SEQ, DP), jnp.float32) for _ in range(NBUF)],
        [pltpu.SemaphoreType.DMA for _ in range(NBUF)],
        [pltpu.SemaphoreType.DMA for _ in range(NBUF)],
        [pltpu.SemaphoreType.DMA for _ in range(NBUF)],
    ],
    compiler_params=pltpu.CompilerParams(use_tc_tiling_on_sc=True),
)
def _emb_lookup(ids_hbm, table_hbm, out_hbm, idx_v, rows_v,
                idx_sem, gat_sem, st_sem):
    wid = lax.axis_index("s") * NC + lax.axis_index("c")
    base = wid * R_PER_W

    def idx_load(i, b):
        pltpu.async_copy(ids_hbm.at[base + i], idx_v[b], idx_sem[b])

    for b in range(NBUF):
        idx_load(b, b)

    def step(g, carry):
        for b in range(NBUF):
            i = g * NBUF + b
            pltpu.make_async_copy(ids_hbm.at[base], idx_v[b],
                                  idx_sem[b]).wait()
            @pl.when(g > 0)
            def _():
                pltpu.make_async_copy(rows_v[b],
                                      out_hbm.at[base], st_sem[b]).wait()
            pltpu.async_copy(table_hbm.at[idx_v[b]], rows_v[b], gat_sem[b])
            pltpu.make_async_copy(table_hbm.at[idx_v[b]], rows_v[b],
                                  gat_sem[b]).wait()
            pltpu.async_copy(rows_v[b], out_hbm.at[base + i], st_sem[b])
            @pl.when(i + NBUF < R_PER_W)
            def _():
                idx_load(i + NBUF, b)
        return carry

    lax.fori_loop(0, R_PER_W // NBUF, step, 0)

    for b in range(NBUF):
        pltpu.make_async_copy(rows_v[b], out_hbm.at[base], st_sem[b]).wait()


def kernel(input_ids, table):
    table_p = jnp.tile(table, (1, DP // D))
    out_p = _emb_lookup(input_ids, table_p)
    return out_p[:, :, :D]


# speedup vs baseline: 1.1450x; 1.1450x over previous
import functools

import jax
import jax.numpy as jnp
from jax import lax
from jax.experimental import pallas as pl
from jax.experimental.pallas import tpu as pltpu
from jax.experimental.pallas import tpu_sc as plsc

NROW = 4096
SEQ = 200
D = 64
DP = 128
NC = 2
NS = 16
NW = NC * NS
R_PER_W = NROW // NW
NBUF = 4

_mesh = plsc.VectorSubcoreMesh(core_axis_name="c", subcore_axis_name="s")


@functools.partial(
    pl.kernel,
    mesh=_mesh,
    out_type=jax.ShapeDtypeStruct((NROW, SEQ, DP), jnp.float32),
    scratch_types=[
        [pltpu.VMEM((SEQ,), jnp.int32) for _ in range(NBUF)],
        [pltpu.VMEM((SEQ, DP), jnp.float32) for _ in range(NBUF)],
        [pltpu.SemaphoreType.DMA for _ in range(NBUF)],
        [pltpu.SemaphoreType.DMA for _ in range(NBUF)],
        [pltpu.SemaphoreType.DMA for _ in range(NBUF)],
    ],
    compiler_params=pltpu.CompilerParams(use_tc_tiling_on_sc=True),
)
def _emb_lookup(ids_hbm, table_hbm, out_hbm, idx_v, rows_v,
                idx_sem, gat_sem, st_sem):
    wid = lax.axis_index("s") * NC + lax.axis_index("c")
    base = wid * R_PER_W

    def idx_load(i, b):
        pltpu.async_copy(ids_hbm.at[base + i], idx_v[b], idx_sem[b])

    for b in range(NBUF):
        idx_load(b, b)

    def step(g, carry):
        for b in range(NBUF):
            i = g * NBUF + b
            pltpu.make_async_copy(ids_hbm.at[base], idx_v[b],
                                  idx_sem[b]).wait()
            @pl.when(g > 0)
            def _():
                pltpu.make_async_copy(rows_v[b],
                                      out_hbm.at[base], st_sem[b]).wait()
            pltpu.async_copy(table_hbm.at[idx_v[b]], rows_v[b], gat_sem[b])
            pltpu.make_async_copy(table_hbm.at[idx_v[b]], rows_v[b],
                                  gat_sem[b]).wait()
            pltpu.async_copy(rows_v[b], out_hbm.at[base + i], st_sem[b])
            @pl.when(i + NBUF < R_PER_W)
            def _():
                idx_load(i + NBUF, b)
        return carry

    lax.fori_loop(0, R_PER_W // NBUF, step, 0)

    for b in range(NBUF):
        pltpu.make_async_copy(rows_v[b], out_hbm.at[base], st_sem[b]).wait()


def kernel(input_ids, table):
    table_p = jnp.pad(table, ((0, 0), (0, DP - D)))
    out_p = _emb_lookup(input_ids, table_p)
    return out_p[:, :, :D]
